# Initial kernel scaffold; baseline (speedup 1.0000x reference)
import jax
import jax.numpy as jnp
from jax.experimental import pallas as pl


def kernel(x, scalars, edge_index, W1, b1, W2, b2):
    raise NotImplementedError("write your pallas kernel here")


# speedup vs baseline: 3.4413x; 3.4413x over previous
import functools

import jax
import jax.numpy as jnp
from jax import lax
from jax.experimental import pallas as pl
from jax.experimental.pallas import tpu as pltpu
from jax.experimental.pallas import tpu_sc as plsc

N = 10000
E = 320000
D = 128
NC = 1
NS = 16
NW = NC * NS
CHUNK = 128
NBUF = 2
NI = 2 * NBUF
CHUNKS_PER_TILE = 160
E_PAD = NW * CHUNKS_PER_TILE * CHUNK
ACC_ROWS = 10240
ROWS_PER_TILE = ACC_ROWS // NS


def _mlp_body(x_ref, w1a_ref, w1b_ref, s_ref, b1_ref, w2_ref, b2_ref, o_ref):
    b1eff = b1_ref[...] + jnp.dot(s_ref[...], w1b_ref[...],
                                  preferred_element_type=jnp.float32)
    h = jnp.dot(x_ref[...], w1a_ref[...], preferred_element_type=jnp.float32)
    h = jnp.maximum(h + b1eff, 0.0)
    o = jnp.dot(h, w2_ref[...], preferred_element_type=jnp.float32)
    o_ref[...] = o + b2_ref[...]


def _node_mlp(x, scalars, W1, b1, W2, b2):
    blk = 1000
    grid = N // blk
    full = lambda shape: pl.BlockSpec(shape, lambda i: (0,) * len(shape))
    return pl.pallas_call(
        _mlp_body,
        grid=(grid,),
        in_specs=[
            pl.BlockSpec((blk, D), lambda i: (i, 0)),
            full((D, D)),
            full((4, D)),
            full((1, 4)),
            full((1, D)),
            full((D, D)),
            full((1, D)),
        ],
        out_specs=pl.BlockSpec((blk, D), lambda i: (i, 0)),
        out_shape=jax.ShapeDtypeStruct((N, D), jnp.float32),
    )(x, W1[:D], W1[D:], scalars, b1.reshape(1, D), W2, b2.reshape(1, D))


def _idx_copy(idx_hbm, wid, chunk, idx_v, q, sem):
    return pltpu.make_async_copy(idx_hbm.at[wid, chunk], idx_v.at[q], sem)


def _sc_body(msg_hbm, idx_hbm, outp_hbm, idx_v, bufs, acc,
             semi, semr):
    s = lax.axis_index("s")
    wid = s

    for q in range(NI):
        _idx_copy(idx_hbm, wid, q, idx_v, q, semi[q]).start()

    zero16 = jnp.zeros((16,), jnp.float32)

    @pl.loop(0, CHUNK)
    def _zero_rows(i):
        for j in range(D // 16):
            bufs[0][i, pl.ds(j * 16, 16)] = zero16

    base = s * ROWS_PER_TILE
    for k in range(ROWS_PER_TILE // CHUNK):
        pltpu.sync_copy(bufs[0], acc.at[pl.ds(base + k * CHUNK, CHUNK)])
    plsc.subcore_barrier()

    for b in range(NBUF):
        _idx_copy(idx_hbm, wid, b, idx_v, b, semi[b]).wait()
        pltpu.async_copy(msg_hbm.at[idx_v.at[b, 0]], bufs[b], semr[b])

    @pl.loop(0, CHUNKS_PER_TILE, step=NI)
    def _edges(g):
        for i in range(NI):
            chunk = g + i
            b = i % NBUF
            pltpu.make_async_copy(msg_hbm.at[idx_v.at[i, 0]],
                                  bufs[b], semr[b]).wait()
            pltpu.sync_copy(bufs[b], acc.at[idx_v.at[i, 1]], add=True)

            @pl.when(chunk + NI < CHUNKS_PER_TILE)
            def _():
                _idx_copy(idx_hbm, wid, chunk + NI, idx_v, i, semi[i]).start()

            q2 = (i + NBUF) % NI

            @pl.when(chunk + NBUF < CHUNKS_PER_TILE)
            def _():
                _idx_copy(idx_hbm, wid, chunk + NBUF, idx_v, q2,
                          semi[q2]).wait()
                pltpu.async_copy(msg_hbm.at[idx_v.at[q2, 0]], bufs[b],
                                 semr[b])

    plsc.subcore_barrier()

    pltpu.sync_copy(acc.at[pl.ds(base, ROWS_PER_TILE)],
                    outp_hbm.at[pl.ds(base, ROWS_PER_TILE)])


def _sc_aggregate(node_msg, idx_t):
    mesh = plsc.VectorSubcoreMesh(core_axis_name="c", subcore_axis_name="s",
                                  num_cores=NC)
    k = pl.kernel(
        _sc_body,
        mesh=mesh,
        out_type=jax.ShapeDtypeStruct((ACC_ROWS, D), jnp.float32),
        scratch_types=[
            pltpu.VMEM((NI, 2, CHUNK), jnp.int32),
            [pltpu.VMEM((CHUNK, D), jnp.float32)] * NBUF,
            pltpu.VMEM_SHARED((ACC_ROWS, D), jnp.float32),
            [pltpu.SemaphoreType.DMA] * NI,
            [pltpu.SemaphoreType.DMA] * NBUF,
        ],
    )
    return k(node_msg, idx_t)


def kernel(x, scalars, edge_index, W1, b1, W2, b2):
    node_msg = _node_mlp(x, scalars, W1, b1, W2, b2)

    src = edge_index[0].astype(jnp.int32)
    dst = edge_index[1].astype(jnp.int32)
    pad = E_PAD - E
    src_t = jnp.concatenate([src, jnp.zeros((pad,), jnp.int32)])
    dst_t = jnp.concatenate([dst, jnp.full((pad,), ACC_ROWS - 1, jnp.int32)])
    idx_t = jnp.stack([src_t.reshape(NW, CHUNKS_PER_TILE, CHUNK),
                       dst_t.reshape(NW, CHUNKS_PER_TILE, CHUNK)], axis=2)

    return _sc_aggregate(node_msg, idx_t)[:N]


# speedup vs baseline: 11.9572x; 3.4746x over previous
import functools

import jax
import jax.numpy as jnp
from jax import lax
from jax.experimental import pallas as pl
from jax.experimental.pallas import tpu as pltpu
from jax.experimental.pallas import tpu_sc as plsc

N = 10000
E = 320000
D = 128
NC = 2
NS = 16
CHUNK = 128
NBUF = 4
NI = 2 * NBUF
CHUNKS_PER_TILE = 160
E_PAD = NS * CHUNKS_PER_TILE * CHUNK
ACC_ROWS = 10240
HALF = ACC_ROWS // NC
ROWS_PER_TILE = HALF // NS


def _mlp_body(x_ref, w1a_ref, w1b_ref, s_ref, b1_ref, w2_ref, b2_ref, o_ref):
    b1eff = b1_ref[...] + jnp.dot(s_ref[...], w1b_ref[...],
                                  preferred_element_type=jnp.float32)
    h = jnp.dot(x_ref[...], w1a_ref[...], preferred_element_type=jnp.float32)
    h = jnp.maximum(h + b1eff, 0.0)
    o = jnp.dot(h, w2_ref[...], preferred_element_type=jnp.float32)
    o_ref[...] = o + b2_ref[...]


def _node_mlp(x, scalars, W1, b1, W2, b2):
    blk = 1000
    grid = N // blk
    full = lambda shape: pl.BlockSpec(shape, lambda i: (0,) * len(shape))
    return pl.pallas_call(
        _mlp_body,
        grid=(grid,),
        in_specs=[
            pl.BlockSpec((blk, D), lambda i: (i, 0)),
            full((D, D)),
            full((4, D)),
            full((1, 4)),
            full((1, D)),
            full((D, D)),
            full((1, D)),
        ],
        out_specs=pl.BlockSpec((blk, D), lambda i: (i, 0)),
        out_shape=jax.ShapeDtypeStruct((N, D), jnp.float32),
    )(x, W1[:D], W1[D:], scalars, b1.reshape(1, D), W2, b2.reshape(1, D))


def _idx_copy(idx_hbm, wid, chunk, idx_v, q, sem):
    return pltpu.make_async_copy(idx_hbm.at[wid, chunk], idx_v.at[q], sem)


def _remap(idx_v, q, lo):
    for j in range(CHUNK // 16):
        sl = pl.ds(j * 16, 16)
        srcv = idx_v[q, 0, sl]
        dl = idx_v[q, 1, sl] - lo
        ok = (dl >= 0) & (dl < HALF)
        neg1 = jnp.full((16,), -1, jnp.int32)
        idx_v[q, 0, sl] = jnp.where(ok, srcv, neg1)
        idx_v[q, 1, sl] = jnp.where(ok, dl, neg1)


def _gather_idx(idx_v, q):
    return plsc.Indices(idx_v.at[q, 0], ignored_value=-1)


def _scatter_idx(idx_v, q):
    return plsc.Indices(idx_v.at[q, 1], ignored_value=-1)


def _sc_body(msg_hbm, idx_hbm, zeros_hbm, out_hbm, idx_v, bufs, acc,
             semi, semr):
    c = lax.axis_index("c")
    s = lax.axis_index("s")
    lo = c * HALF

    for q in range(NI):
        _idx_copy(idx_hbm, s, q, idx_v, q, semi[q]).start()

    base = s * ROWS_PER_TILE
    pltpu.sync_copy(zeros_hbm, acc.at[pl.ds(base, ROWS_PER_TILE)])
    plsc.subcore_barrier()

    for b in range(NBUF):
        _idx_copy(idx_hbm, s, b, idx_v, b, semi[b]).wait()
        _remap(idx_v, b, lo)
        pltpu.async_copy(msg_hbm.at[_gather_idx(idx_v, b)], bufs[b], semr[b])

    @pl.loop(0, CHUNKS_PER_TILE, step=NI)
    def _edges(g):
        for i in range(NI):
            chunk = g + i
            b = i % NBUF
            pltpu.make_async_copy(msg_hbm.at[_gather_idx(idx_v, i)],
                                  bufs[b], semr[b]).wait()
            pltpu.sync_copy(bufs[b], acc.at[_scatter_idx(idx_v, i)],
                            add=True)

            @pl.when(chunk + NI < CHUNKS_PER_TILE)
            def _():
                _idx_copy(idx_hbm, s, chunk + NI, idx_v, i, semi[i]).start()

            q2 = (i + NBUF) % NI

            @pl.when(chunk + NBUF < CHUNKS_PER_TILE)
            def _():
                _idx_copy(idx_hbm, s, chunk + NBUF, idx_v, q2,
                          semi[q2]).wait()
                _remap(idx_v, q2, lo)
                pltpu.async_copy(msg_hbm.at[_gather_idx(idx_v, q2)], bufs[b],
                                 semr[b])

    plsc.subcore_barrier()

    pltpu.sync_copy(acc.at[pl.ds(base, ROWS_PER_TILE)],
                    out_hbm.at[pl.ds(lo + base, ROWS_PER_TILE)])


def _sc_aggregate(node_msg, idx_t, zeros):
    mesh = plsc.VectorSubcoreMesh(core_axis_name="c", subcore_axis_name="s",
                                  num_cores=NC)
    k = pl.kernel(
        _sc_body,
        mesh=mesh,
        out_type=jax.ShapeDtypeStruct((ACC_ROWS, D), jnp.float32),
        scratch_types=[
            pltpu.VMEM((NI, 2, CHUNK), jnp.int32),
            [pltpu.VMEM((CHUNK, D), jnp.float32)] * NBUF,
            pltpu.VMEM_SHARED((HALF, D), jnp.float32),
            [pltpu.SemaphoreType.DMA] * NI,
            [pltpu.SemaphoreType.DMA] * NBUF,
        ],
    )
    return k(node_msg, idx_t, zeros)


def kernel(x, scalars, edge_index, W1, b1, W2, b2):
    node_msg = _node_mlp(x, scalars, W1, b1, W2, b2)

    src = edge_index[0].astype(jnp.int32)
    dst = edge_index[1].astype(jnp.int32)
    pad = E_PAD - E
    fill = jnp.full((pad,), -1, jnp.int32)
    src_t = jnp.concatenate([src, fill])
    dst_t = jnp.concatenate([dst, fill])
    idx_t = jnp.stack([src_t.reshape(NS, CHUNKS_PER_TILE, CHUNK),
                       dst_t.reshape(NS, CHUNKS_PER_TILE, CHUNK)], axis=2)

    zeros = jnp.zeros((ROWS_PER_TILE, D), jnp.float32)
    return _sc_aggregate(node_msg, idx_t, zeros)[:N]
